# Initial kernel scaffold; baseline (speedup 1.0000x reference)
import functools

import jax
import jax.numpy as jnp
from jax import lax
from jax.experimental import pallas as pl
from jax.experimental.pallas import tpu as pltpu

_LEVELS = [(1024, 8), (512, 16), (256, 32), (128, 64)]


def _sa_body(P_ref, W_ref, b_ref, q_ref, xT_ref, o_ref, *rest, N, Sc, k, OUT,
             last):
    if last:
        po_ref = rest[0]
        pre_ref, r_ref, acc_ref = rest[1:]
    else:
        pre_ref, r_ref, acc_ref = rest

    P = P_ref[0]
    W = W_ref[...]
    pre_ref[...] = (jnp.dot(P, W, preferred_element_type=jnp.float32)
                    + b_ref[...])
    X = xT_ref[0]
    x2 = jnp.sum(X * X, axis=0, keepdims=True)
    q = q_ref[0]
    r = x2 - 2.0 * (q[:, 0:1] * X[0:1, :]
                    + q[:, 1:2] * X[1:2, :]
                    + q[:, 2:3] * X[2:3, :])
    r_ref[...] = r
    acc_ref[...] = jnp.full((Sc, OUT), -jnp.inf, jnp.float32)
    iota = lax.broadcasted_iota(jnp.int32, (Sc, N), 1)

    def body(t, carry):
        rr = r_ref[...]
        m = jnp.min(rr, axis=1, keepdims=True)
        cand = jnp.where(rr <= m, iota, N)
        am = jnp.min(cand, axis=1, keepdims=True)
        onehot = iota == am
        r_ref[...] = jnp.where(onehot, jnp.inf, rr)
        g = jnp.dot(onehot.astype(jnp.float32), pre_ref[...],
                    preferred_element_type=jnp.float32)
        acc_ref[...] = jnp.maximum(acc_ref[...], g)
        return carry

    lax.fori_loop(0, k, body, 0)
    qx = jnp.dot(q, W[0:3, :], preferred_element_type=jnp.float32)
    res = jnp.maximum(acc_ref[...] - qx, 0.0)
    o_ref[0] = res
    if last:
        po_ref[0] = jnp.max(res, axis=0, keepdims=True)


def _sa_level(xyz, feats, W, b, n_s, k, Sc, last=False):
    B, N, C = feats.shape
    IN = 3 + C
    OUT = W.shape[1]
    stride = N // n_s
    q = xyz[:, ::stride, :]
    P = jnp.concatenate([xyz, feats], axis=-1)
    xT = jnp.transpose(xyz, (0, 2, 1))
    b2 = b.reshape(1, OUT)
    NC = n_s // Sc

    out_shapes = [jax.ShapeDtypeStruct((B, n_s, OUT), jnp.float32)]
    out_specs = [pl.BlockSpec((1, Sc, OUT), lambda bb, cc: (bb, cc, 0))]
    if last:
        out_shapes.append(jax.ShapeDtypeStruct((B, 1, OUT), jnp.float32))
        out_specs.append(pl.BlockSpec((1, 1, OUT), lambda bb, cc: (bb, 0, 0)))

    outs = pl.pallas_call(
        functools.partial(_sa_body, N=N, Sc=Sc, k=k, OUT=OUT, last=last),
        grid=(B, NC),
        in_specs=[
            pl.BlockSpec((1, N, IN), lambda bb, cc: (bb, 0, 0)),
            pl.BlockSpec((IN, OUT), lambda bb, cc: (0, 0)),
            pl.BlockSpec((1, OUT), lambda bb, cc: (0, 0)),
            pl.BlockSpec((1, Sc, 3), lambda bb, cc: (bb, cc, 0)),
            pl.BlockSpec((1, 3, N), lambda bb, cc: (bb, 0, 0)),
        ],
        out_specs=out_specs,
        out_shape=out_shapes,
        scratch_shapes=[
            pltpu.VMEM((N, OUT), jnp.float32),
            pltpu.VMEM((Sc, N), jnp.float32),
            pltpu.VMEM((Sc, OUT), jnp.float32),
        ],
    )(P, W, b2, q, xT)
    if last:
        return q, outs[0], outs[1]
    return q, outs[0]


def _cls_body(p_ref, W0_ref, b0_ref, W1_ref, b1_ref, W2_ref, b2_ref, o_ref):
    p = p_ref[...]
    h = jnp.maximum(jnp.dot(p, W0_ref[...],
                            preferred_element_type=jnp.float32)
                    + b0_ref[...], 0.0)
    h = jnp.maximum(jnp.dot(h, W1_ref[...],
                            preferred_element_type=jnp.float32)
                    + b1_ref[...], 0.0)
    o_ref[...] = (jnp.dot(h, W2_ref[...], preferred_element_type=jnp.float32)
                  + b2_ref[...])


def kernel(x, W0, b0, W1, b1, W2, b2, W3, b3, Wc0, bc0, Wc1, bc1, Wc2, bc2):
    xyz = jnp.transpose(x[:, :3, :], (0, 2, 1))
    normal = jnp.transpose(x[:, 3:, :], (0, 2, 1))
    feats = jnp.concatenate([xyz, normal], axis=-1)

    chunks = [512, 512, 256, 128]
    params = [(W0, b0), (W1, b1), (W2, b2), (W3, b3)]
    pooled = None
    for i, ((n_s, k), (W, b)) in enumerate(zip(_LEVELS, params)):
        last = i == len(_LEVELS) - 1
        if last:
            xyz, feats, pooled = _sa_level(xyz, feats, W, b, n_s, k,
                                           chunks[i], last=True)
        else:
            xyz, feats = _sa_level(xyz, feats, W, b, n_s, k, chunks[i])

    B = x.shape[0]
    pooled = pooled.reshape(B, -1)
    logits = pl.pallas_call(
        _cls_body,
        out_shape=jax.ShapeDtypeStruct((B, Wc2.shape[1]), jnp.float32),
    )(pooled, Wc0, bc0.reshape(1, -1), Wc1, bc1.reshape(1, -1),
      Wc2, bc2.reshape(1, -1))
    new_points = jnp.transpose(feats, (0, 2, 1))
    return logits, new_points


# speedup vs baseline: 7.2137x; 7.2137x over previous
import functools

import jax
import jax.numpy as jnp
from jax import lax
from jax.experimental import pallas as pl
from jax.experimental.pallas import tpu as pltpu

_LEVELS = [(1024, 8), (512, 16), (256, 32), (128, 64)]


def _sa_body(P_ref, W_ref, b_ref, q_ref, xT_ref, o_ref, *rest, N, Sc, k, OUT,
             last):
    if last:
        po_ref = rest[0]
        pre_ref, r_ref, acc_ref = rest[1:]
    else:
        pre_ref, r_ref, acc_ref = rest

    P = P_ref[0]
    W = W_ref[...]
    pre_ref[...] = (jnp.dot(P, W, preferred_element_type=jnp.float32,
                            precision=lax.Precision.HIGHEST)
                    + b_ref[...])
    X = xT_ref[0]
    x2 = jnp.sum(X * X, axis=0, keepdims=True)
    q = q_ref[0]
    Xb = X.astype(jnp.bfloat16).astype(jnp.float32)
    qb = q.astype(jnp.bfloat16).astype(jnp.float32)
    r = x2 - 2.0 * (qb[:, 0:1] * Xb[0:1, :]
                    + qb[:, 1:2] * Xb[1:2, :]
                    + qb[:, 2:3] * Xb[2:3, :])
    r_ref[...] = r
    acc_ref[...] = jnp.full((Sc, OUT), -jnp.inf, jnp.float32)
    iota = lax.broadcasted_iota(jnp.int32, (Sc, N), 1)

    def body(t, carry):
        rr = r_ref[...]
        m = jnp.min(rr, axis=1, keepdims=True)
        cand = jnp.where(rr <= m, iota, N)
        am = jnp.min(cand, axis=1, keepdims=True)
        onehot = iota == am
        r_ref[...] = jnp.where(onehot, jnp.inf, rr)
        g = jnp.dot(onehot.astype(jnp.float32), pre_ref[...],
                    preferred_element_type=jnp.float32,
                    precision=lax.Precision.HIGHEST)
        acc_ref[...] = jnp.maximum(acc_ref[...], g)
        return carry

    lax.fori_loop(0, k, body, 0)
    qx = jnp.dot(q, W[0:3, :], preferred_element_type=jnp.float32,
                 precision=lax.Precision.HIGHEST)
    res = jnp.maximum(acc_ref[...] - qx, 0.0)
    o_ref[0] = res
    if last:
        po_ref[0] = jnp.max(res, axis=0, keepdims=True)


def _sa_level(xyz, feats, W, b, n_s, k, Sc, last=False):
    B, N, C = feats.shape
    IN = 3 + C
    OUT = W.shape[1]
    stride = N // n_s
    q = xyz[:, ::stride, :]
    P = jnp.concatenate([xyz, feats], axis=-1)
    xT = jnp.transpose(xyz, (0, 2, 1))
    b2 = b.reshape(1, OUT)
    NC = n_s // Sc

    out_shapes = [jax.ShapeDtypeStruct((B, n_s, OUT), jnp.float32)]
    out_specs = [pl.BlockSpec((1, Sc, OUT), lambda bb, cc: (bb, cc, 0))]
    if last:
        out_shapes.append(jax.ShapeDtypeStruct((B, 1, OUT), jnp.float32))
        out_specs.append(pl.BlockSpec((1, 1, OUT), lambda bb, cc: (bb, 0, 0)))

    outs = pl.pallas_call(
        functools.partial(_sa_body, N=N, Sc=Sc, k=k, OUT=OUT, last=last),
        grid=(B, NC),
        in_specs=[
            pl.BlockSpec((1, N, IN), lambda bb, cc: (bb, 0, 0)),
            pl.BlockSpec((IN, OUT), lambda bb, cc: (0, 0)),
            pl.BlockSpec((1, OUT), lambda bb, cc: (0, 0)),
            pl.BlockSpec((1, Sc, 3), lambda bb, cc: (bb, cc, 0)),
            pl.BlockSpec((1, 3, N), lambda bb, cc: (bb, 0, 0)),
        ],
        out_specs=out_specs,
        out_shape=out_shapes,
        scratch_shapes=[
            pltpu.VMEM((N, OUT), jnp.float32),
            pltpu.VMEM((Sc, N), jnp.float32),
            pltpu.VMEM((Sc, OUT), jnp.float32),
        ],
    )(P, W, b2, q, xT)
    if last:
        return q, outs[0], outs[1]
    return q, outs[0]


def _cls_body(p_ref, W0_ref, b0_ref, W1_ref, b1_ref, W2_ref, b2_ref, o_ref):
    p = p_ref[...]
    h = jnp.maximum(jnp.dot(p, W0_ref[...],
                            preferred_element_type=jnp.float32)
                    + b0_ref[...], 0.0)
    h = jnp.maximum(jnp.dot(h, W1_ref[...],
                            preferred_element_type=jnp.float32)
                    + b1_ref[...], 0.0)
    o_ref[...] = (jnp.dot(h, W2_ref[...], preferred_element_type=jnp.float32)
                  + b2_ref[...])


def kernel(x, W0, b0, W1, b1, W2, b2, W3, b3, Wc0, bc0, Wc1, bc1, Wc2, bc2):
    xyz = jnp.transpose(x[:, :3, :], (0, 2, 1))
    normal = jnp.transpose(x[:, 3:, :], (0, 2, 1))
    feats = jnp.concatenate([xyz, normal], axis=-1)

    chunks = [512, 512, 256, 128]
    params = [(W0, b0), (W1, b1), (W2, b2), (W3, b3)]
    pooled = None
    for i, ((n_s, k), (W, b)) in enumerate(zip(_LEVELS, params)):
        last = i == len(_LEVELS) - 1
        if last:
            xyz, feats, pooled = _sa_level(xyz, feats, W, b, n_s, k,
                                           chunks[i], last=True)
        else:
            xyz, feats = _sa_level(xyz, feats, W, b, n_s, k, chunks[i])

    B = x.shape[0]
    pooled = pooled.reshape(B, -1)
    logits = pl.pallas_call(
        _cls_body,
        out_shape=jax.ShapeDtypeStruct((B, Wc2.shape[1]), jnp.float32),
    )(pooled, Wc0, bc0.reshape(1, -1), Wc1, bc1.reshape(1, -1),
      Wc2, bc2.reshape(1, -1))
    new_points = jnp.transpose(feats, (0, 2, 1))
    return logits, new_points


# speedup vs baseline: 17.5217x; 2.4289x over previous
import functools

import jax
import jax.numpy as jnp
from jax import lax
from jax.experimental import pallas as pl
from jax.experimental.pallas import tpu as pltpu
from jax.experimental.pallas import tpu_sc as plsc

_LEVELS = [(1024, 8), (512, 16), (256, 32), (128, 64)]


def _sel_body(P_ref, W_ref, b_ref, q_ref, xT_ref, idx_ref, pre_ref, qx_ref,
              r_ref, *, N, Sc, k, OUTP):
    P = P_ref[0]
    W = W_ref[...]
    pre = (jnp.dot(P, W, preferred_element_type=jnp.float32,
                   precision=lax.Precision.HIGHEST)
           + b_ref[...])
    OUT = W.shape[1]
    if OUTP > OUT:
        pre = jnp.concatenate(
            [pre, jnp.zeros((N, OUTP - OUT), jnp.float32)], axis=1)
    pre_ref[0] = pre
    q = q_ref[0]
    qx_ref[0] = jnp.dot(q, W[0:3, :], preferred_element_type=jnp.float32,
                        precision=lax.Precision.HIGHEST)
    X = xT_ref[0]
    x2 = jnp.sum(X * X, axis=0, keepdims=True)
    Xb = X.astype(jnp.bfloat16).astype(jnp.float32)
    qb = q.astype(jnp.bfloat16).astype(jnp.float32)
    r_ref[...] = x2 - 2.0 * (qb[:, 0:1] * Xb[0:1, :]
                             + qb[:, 1:2] * Xb[1:2, :]
                             + qb[:, 2:3] * Xb[2:3, :])
    iota = lax.broadcasted_iota(jnp.int32, (Sc, N), 1)
    bN = pl.program_id(0) * N
    cols = []
    for _ in range(k):
        rr = r_ref[...]
        m = jnp.min(rr, axis=1, keepdims=True)
        cand = jnp.where(rr <= m, iota, N)
        am = jnp.min(cand, axis=1, keepdims=True)
        r_ref[...] = jnp.where(iota == am, jnp.inf, rr)
        cols.append(am)
    idx_ref[0] = jnp.concatenate(cols, axis=1) + bN


def _select(xyz, feats, W, b, n_s, k, Sc):
    B, N, C = feats.shape
    IN = 3 + C
    OUT = W.shape[1]
    stride = N // n_s
    q = xyz[:, ::stride, :]
    P = jnp.concatenate([xyz, feats], axis=-1)
    xT = jnp.transpose(xyz, (0, 2, 1))
    NC = n_s // Sc

    OUTP = max(OUT, 128)
    idx, pre, qx = pl.pallas_call(
        functools.partial(_sel_body, N=N, Sc=Sc, k=k, OUTP=OUTP),
        grid=(B, NC),
        in_specs=[
            pl.BlockSpec((1, N, IN), lambda bb, cc: (bb, 0, 0)),
            pl.BlockSpec((IN, OUT), lambda bb, cc: (0, 0)),
            pl.BlockSpec((1, OUT), lambda bb, cc: (0, 0)),
            pl.BlockSpec((1, Sc, 3), lambda bb, cc: (bb, cc, 0)),
            pl.BlockSpec((1, 3, N), lambda bb, cc: (bb, 0, 0)),
        ],
        out_specs=[
            pl.BlockSpec((1, Sc, k), lambda bb, cc: (bb, cc, 0)),
            pl.BlockSpec((1, N, OUTP), lambda bb, cc: (bb, 0, 0)),
            pl.BlockSpec((1, Sc, OUT), lambda bb, cc: (bb, cc, 0)),
        ],
        out_shape=[
            jax.ShapeDtypeStruct((B, n_s, k), jnp.int32),
            jax.ShapeDtypeStruct((B, N, OUTP), jnp.float32),
            jax.ShapeDtypeStruct((B, n_s, OUT), jnp.float32),
        ],
        scratch_shapes=[pltpu.VMEM((Sc, N), jnp.float32)],
    )(P, W, b.reshape(1, OUT), q, xT)
    return q, idx, pre, qx


def _gather_max_sc(table, idxf, qx, k):
    R, OUTP = table.shape
    Q, OUT = qx.shape
    G = max(1, 65536 // (k * OUTP))
    info = plsc.get_sparse_core_info()
    NC, NS = info.num_cores, info.num_subcores
    NW = NC * NS
    qpw = Q // NW
    ngroups = qpw // G
    nco = OUT // 16
    mesh = plsc.VectorSubcoreMesh(core_axis_name="c", subcore_axis_name="s")

    @functools.partial(
        pl.kernel, mesh=mesh,
        out_type=jax.ShapeDtypeStruct((Q, OUT), jnp.float32),
        scratch_types=[
            pltpu.VMEM((G * k,), jnp.int32),
            pltpu.VMEM((G * k, OUTP), jnp.float32),
            pltpu.VMEM((G, OUT), jnp.float32),
            pltpu.VMEM((G, OUT), jnp.float32),
            pltpu.SemaphoreType.DMA,
        ],
    )
    def kern(table_hbm, idx_hbm, qx_hbm, out_hbm, idx_v, rows_v, qx_v, out_v,
             sem):
        wid = lax.axis_index("s") * NC + lax.axis_index("c")
        qbase0 = wid * qpw

        def group(g, carry):
            qb = qbase0 + g * G
            pltpu.sync_copy(idx_hbm.at[pl.ds(qb * k, G * k)], idx_v)
            pltpu.async_copy(table_hbm.at[idx_v], rows_v, sem).wait()
            pltpu.sync_copy(qx_hbm.at[pl.ds(qb, G)], qx_v)

            def per_query(qi, c2):
                base = qi * k
                accs = [rows_v[base, pl.ds(co * 16, 16)] for co in range(nco)]

                def per_j(j, accs):
                    return tuple(
                        jnp.maximum(a, rows_v[base + j, pl.ds(co * 16, 16)])
                        for co, a in enumerate(accs))

                accs = lax.fori_loop(1, k, per_j, tuple(accs))
                for co, a in enumerate(accs):
                    sl = pl.ds(co * 16, 16)
                    out_v[qi, sl] = jnp.maximum(a - qx_v[qi, sl], 0.0)
                return c2

            lax.fori_loop(0, G, per_query, 0)
            pltpu.sync_copy(out_v, out_hbm.at[pl.ds(qb, G)])
            return carry

        lax.fori_loop(0, ngroups, group, 0)

    return kern(table, idxf, qx)


def _cls_body(f_ref, W0_ref, b0_ref, W1_ref, b1_ref, W2_ref, b2_ref, o_ref):
    pooled = jnp.max(f_ref[...], axis=1)
    h = jnp.maximum(jnp.dot(pooled, W0_ref[...],
                            preferred_element_type=jnp.float32,
                            precision=lax.Precision.HIGHEST)
                    + b0_ref[...], 0.0)
    h = jnp.maximum(jnp.dot(h, W1_ref[...],
                            preferred_element_type=jnp.float32,
                            precision=lax.Precision.HIGHEST)
                    + b1_ref[...], 0.0)
    o_ref[...] = (jnp.dot(h, W2_ref[...], preferred_element_type=jnp.float32,
                          precision=lax.Precision.HIGHEST)
                  + b2_ref[...])


def kernel(x, W0, b0, W1, b1, W2, b2, W3, b3, Wc0, bc0, Wc1, bc1, Wc2, bc2):
    B = x.shape[0]
    xyz = jnp.transpose(x[:, :3, :], (0, 2, 1))
    normal = jnp.transpose(x[:, 3:, :], (0, 2, 1))
    feats = jnp.concatenate([xyz, normal], axis=-1)

    chunks = [512, 512, 256, 128]
    params = [(W0, b0), (W1, b1), (W2, b2), (W3, b3)]
    for i, ((n_s, k), (W, b)) in enumerate(zip(_LEVELS, params)):
        N = feats.shape[1]
        OUT = W.shape[1]
        xyz, idx, pre, qx = _select(xyz, feats, W, b, n_s, k, chunks[i])
        out = _gather_max_sc(pre.reshape(B * N, -1), idx.reshape(-1),
                             qx.reshape(B * n_s, OUT), k)
        feats = out.reshape(B, n_s, OUT)

    logits = pl.pallas_call(
        _cls_body,
        out_shape=jax.ShapeDtypeStruct((B, Wc2.shape[1]), jnp.float32),
    )(feats, Wc0, bc0.reshape(1, -1), Wc1, bc1.reshape(1, -1),
      Wc2, bc2.reshape(1, -1))
    new_points = jnp.transpose(feats, (0, 2, 1))
    return logits, new_points


# speedup vs baseline: 19.3813x; 1.1061x over previous
import functools

import jax
import jax.numpy as jnp
from jax import lax
from jax.experimental import pallas as pl
from jax.experimental.pallas import tpu as pltpu
from jax.experimental.pallas import tpu_sc as plsc

_LEVELS = [(1024, 8), (512, 16), (256, 32), (128, 64)]


def _sel_body(P_ref, W_ref, b_ref, q_ref, xT_ref, idx_ref, pre_ref, qx_ref,
              r_ref, *, N, Sc, k, OUTP):
    P = P_ref[0]
    W = W_ref[...]
    pre = (jnp.dot(P, W, preferred_element_type=jnp.float32,
                   precision=lax.Precision.HIGHEST)
           + b_ref[...])
    OUT = W.shape[1]
    if OUTP > OUT:
        pre = jnp.concatenate(
            [pre, jnp.zeros((N, OUTP - OUT), jnp.float32)], axis=1)
    pre_ref[0] = pre
    q = q_ref[0]
    qx_ref[0] = jnp.dot(q, W[0:3, :], preferred_element_type=jnp.float32,
                        precision=lax.Precision.HIGHEST)
    X = xT_ref[0]
    x2 = jnp.sum(X * X, axis=0, keepdims=True)
    Xb = X.astype(jnp.bfloat16)
    qb = q.astype(jnp.bfloat16)
    r_ref[...] = x2 - 2.0 * jnp.dot(qb, Xb,
                                    preferred_element_type=jnp.float32)
    iota = lax.broadcasted_iota(jnp.int32, (Sc, N), 1).astype(jnp.float32)
    bN = pl.program_id(0) * N
    cols = []
    for _ in range(k):
        rr = r_ref[...]
        m = jnp.min(rr, axis=1, keepdims=True)
        cand = jnp.where(rr <= m, iota, jnp.float32(N))
        am = jnp.min(cand, axis=1, keepdims=True)
        r_ref[...] = jnp.where(iota == am, jnp.inf, rr)
        cols.append(am)
    idx_ref[0] = (jnp.concatenate(cols, axis=1).astype(jnp.int32)
                  + bN)


def _select(xyz, feats, W, b, n_s, k, Sc):
    B, N, C = feats.shape
    IN = 3 + C
    OUT = W.shape[1]
    stride = N // n_s
    q = xyz[:, ::stride, :]
    P = jnp.concatenate([xyz, feats], axis=-1)
    xT = jnp.transpose(xyz, (0, 2, 1))
    NC = n_s // Sc

    OUTP = max(OUT, 128)
    idx, pre, qx = pl.pallas_call(
        functools.partial(_sel_body, N=N, Sc=Sc, k=k, OUTP=OUTP),
        grid=(B, NC),
        in_specs=[
            pl.BlockSpec((1, N, IN), lambda bb, cc: (bb, 0, 0)),
            pl.BlockSpec((IN, OUT), lambda bb, cc: (0, 0)),
            pl.BlockSpec((1, OUT), lambda bb, cc: (0, 0)),
            pl.BlockSpec((1, Sc, 3), lambda bb, cc: (bb, cc, 0)),
            pl.BlockSpec((1, 3, N), lambda bb, cc: (bb, 0, 0)),
        ],
        out_specs=[
            pl.BlockSpec((1, Sc, k), lambda bb, cc: (bb, cc, 0)),
            pl.BlockSpec((1, N, OUTP), lambda bb, cc: (bb, 0, 0)),
            pl.BlockSpec((1, Sc, OUT), lambda bb, cc: (bb, cc, 0)),
        ],
        out_shape=[
            jax.ShapeDtypeStruct((B, n_s, k), jnp.int32),
            jax.ShapeDtypeStruct((B, N, OUTP), jnp.float32),
            jax.ShapeDtypeStruct((B, n_s, OUT), jnp.float32),
        ],
        scratch_shapes=[pltpu.VMEM((Sc, N), jnp.float32)],
    )(P, W, b.reshape(1, OUT), q, xT)
    return q, idx, pre, qx


def _gather_max_sc(table, idxf, qx, k):
    R, OUTP = table.shape
    Q, OUT = qx.shape
    G = max(1, 65536 // (k * OUTP))
    info = plsc.get_sparse_core_info()
    NC, NS = info.num_cores, info.num_subcores
    NW = NC * NS
    qpw = Q // NW
    ngroups = qpw // G
    nco = OUT // 16
    mesh = plsc.VectorSubcoreMesh(core_axis_name="c", subcore_axis_name="s")

    @functools.partial(
        pl.kernel, mesh=mesh,
        out_type=jax.ShapeDtypeStruct((Q, OUT), jnp.float32),
        scratch_types=[
            pltpu.VMEM((G * k,), jnp.int32),
            pltpu.VMEM((G * k, OUTP), jnp.float32),
            pltpu.VMEM((G, OUT), jnp.float32),
            pltpu.VMEM((G, OUT), jnp.float32),
            pltpu.SemaphoreType.DMA,
        ],
    )
    def kern(table_hbm, idx_hbm, qx_hbm, out_hbm, idx_v, rows_v, qx_v, out_v,
             sem):
        wid = lax.axis_index("s") * NC + lax.axis_index("c")
        qbase0 = wid * qpw

        def group(g, carry):
            qb = qbase0 + g * G
            pltpu.sync_copy(idx_hbm.at[pl.ds(qb * k, G * k)], idx_v)
            pltpu.async_copy(table_hbm.at[idx_v], rows_v, sem).wait()
            pltpu.sync_copy(qx_hbm.at[pl.ds(qb, G)], qx_v)

            def per_query(qi, c2):
                base = qi * k

                def per_co(co, c3):
                    sl = pl.ds(co * 16, 16)
                    vals = [rows_v[base + j, sl] for j in range(k)]
                    while len(vals) > 1:
                        nxt = [jnp.maximum(vals[t], vals[t + 1])
                               for t in range(0, len(vals) - 1, 2)]
                        if len(vals) % 2:
                            nxt.append(vals[-1])
                        vals = nxt
                    out_v[qi, sl] = jnp.maximum(vals[0] - qx_v[qi, sl], 0.0)
                    return c3

                return lax.fori_loop(0, nco, per_co, c2)

            lax.fori_loop(0, G, per_query, 0)
            pltpu.sync_copy(out_v, out_hbm.at[pl.ds(qb, G)])
            return carry

        lax.fori_loop(0, ngroups, group, 0)

    return kern(table, idxf, qx)


def _cls_body(f_ref, W0_ref, b0_ref, W1_ref, b1_ref, W2_ref, b2_ref, o_ref):
    pooled = jnp.max(f_ref[...], axis=1)
    h = jnp.maximum(jnp.dot(pooled, W0_ref[...],
                            preferred_element_type=jnp.float32,
                            precision=lax.Precision.HIGHEST)
                    + b0_ref[...], 0.0)
    h = jnp.maximum(jnp.dot(h, W1_ref[...],
                            preferred_element_type=jnp.float32,
                            precision=lax.Precision.HIGHEST)
                    + b1_ref[...], 0.0)
    o_ref[...] = (jnp.dot(h, W2_ref[...], preferred_element_type=jnp.float32,
                          precision=lax.Precision.HIGHEST)
                  + b2_ref[...])


def kernel(x, W0, b0, W1, b1, W2, b2, W3, b3, Wc0, bc0, Wc1, bc1, Wc2, bc2):
    B = x.shape[0]
    xyz = jnp.transpose(x[:, :3, :], (0, 2, 1))
    normal = jnp.transpose(x[:, 3:, :], (0, 2, 1))
    feats = jnp.concatenate([xyz, normal], axis=-1)

    chunks = [1024, 512, 256, 128]
    params = [(W0, b0), (W1, b1), (W2, b2), (W3, b3)]
    for i, ((n_s, k), (W, b)) in enumerate(zip(_LEVELS, params)):
        N = feats.shape[1]
        OUT = W.shape[1]
        xyz, idx, pre, qx = _select(xyz, feats, W, b, n_s, k, chunks[i])
        out = _gather_max_sc(pre.reshape(B * N, -1), idx.reshape(-1),
                             qx.reshape(B * n_s, OUT), k)
        feats = out.reshape(B, n_s, OUT)

    logits = pl.pallas_call(
        _cls_body,
        out_shape=jax.ShapeDtypeStruct((B, Wc2.shape[1]), jnp.float32),
    )(feats, Wc0, bc0.reshape(1, -1), Wc1, bc1.reshape(1, -1),
      Wc2, bc2.reshape(1, -1))
    new_points = jnp.transpose(feats, (0, 2, 1))
    return logits, new_points


# speedup vs baseline: 21.6430x; 1.1167x over previous
import functools

import jax
import jax.numpy as jnp
from jax import lax
from jax.experimental import pallas as pl
from jax.experimental.pallas import tpu as pltpu
from jax.experimental.pallas import tpu_sc as plsc

_LEVELS = [(1024, 8), (512, 16), (256, 32), (128, 64)]


def _sel_body(P_ref, W_ref, b_ref, q_ref, xT_ref, idx_ref, pre_ref, qx_ref,
              r_ref, *, N, Sc, k, OUTP):
    P = P_ref[0]
    W = W_ref[...]
    pre = (jnp.dot(P, W, preferred_element_type=jnp.float32,
                   precision=lax.Precision.HIGHEST)
           + b_ref[...])
    OUT = W.shape[1]
    if OUTP > OUT:
        pre = jnp.concatenate(
            [pre, jnp.zeros((N, OUTP - OUT), jnp.float32)], axis=1)
    pre_ref[0] = pre
    q = q_ref[0]
    qx_ref[0] = jnp.dot(q, W[0:3, :], preferred_element_type=jnp.float32,
                        precision=lax.Precision.HIGHEST)
    X = xT_ref[0]
    x2 = jnp.sum(X * X, axis=0, keepdims=True)
    Xb = X.astype(jnp.bfloat16)
    qb = q.astype(jnp.bfloat16)
    r_ref[...] = x2 - 2.0 * jnp.dot(qb, Xb,
                                    preferred_element_type=jnp.float32)
    iota = lax.broadcasted_iota(jnp.int32, (Sc, N), 1).astype(jnp.float32)
    bN = pl.program_id(0) * N
    cols = []
    for _ in range(k):
        rr = r_ref[...]
        m = jnp.min(rr, axis=1, keepdims=True)
        cand = jnp.where(rr <= m, iota, jnp.float32(N))
        am = jnp.min(cand, axis=1, keepdims=True)
        r_ref[...] = jnp.where(iota == am, jnp.inf, rr)
        cols.append(am)
    idx_ref[0] = (jnp.concatenate(cols, axis=1).astype(jnp.int32)
                  + bN)


def _select(xyz, feats, W, b, n_s, k, Sc):
    B, N, C = feats.shape
    IN = 3 + C
    OUT = W.shape[1]
    stride = N // n_s
    q = xyz[:, ::stride, :]
    P = jnp.concatenate([xyz, feats], axis=-1)
    xT = jnp.transpose(xyz, (0, 2, 1))
    NC = n_s // Sc

    OUTP = max(OUT, 128)
    idx, pre, qx = pl.pallas_call(
        functools.partial(_sel_body, N=N, Sc=Sc, k=k, OUTP=OUTP),
        grid=(B, NC),
        in_specs=[
            pl.BlockSpec((1, N, IN), lambda bb, cc: (bb, 0, 0)),
            pl.BlockSpec((IN, OUT), lambda bb, cc: (0, 0)),
            pl.BlockSpec((1, OUT), lambda bb, cc: (0, 0)),
            pl.BlockSpec((1, Sc, 3), lambda bb, cc: (bb, cc, 0)),
            pl.BlockSpec((1, 3, N), lambda bb, cc: (bb, 0, 0)),
        ],
        out_specs=[
            pl.BlockSpec((1, Sc, k), lambda bb, cc: (bb, cc, 0)),
            pl.BlockSpec((1, N, OUTP), lambda bb, cc: (bb, 0, 0)),
            pl.BlockSpec((1, Sc, OUT), lambda bb, cc: (bb, cc, 0)),
        ],
        out_shape=[
            jax.ShapeDtypeStruct((B, n_s, k), jnp.int32),
            jax.ShapeDtypeStruct((B, N, OUTP), jnp.float32),
            jax.ShapeDtypeStruct((B, n_s, OUT), jnp.float32),
        ],
        scratch_shapes=[pltpu.VMEM((Sc, N), jnp.float32)],
    )(P, W, b.reshape(1, OUT), q, xT)
    return q, idx, pre, qx


def _gather_max_sc(table, idxf, qx, k):
    R, OUTP = table.shape
    Q, OUT = qx.shape
    info = plsc.get_sparse_core_info()
    NC, NS = info.num_cores, info.num_subcores
    NW = NC * NS
    qpw = Q // NW
    if k * OUTP <= 16384:
        SUB = 1
        qpc = max(1, min(16384 // (k * OUTP), qpw))
    else:
        SUB = 2
        qpc = 1
    kc = k // SUB
    nchunks = (qpw // qpc) * SUB
    nco = OUT // 16
    mesh = plsc.VectorSubcoreMesh(core_axis_name="c", subcore_axis_name="s")

    @functools.partial(
        pl.kernel, mesh=mesh,
        out_type=jax.ShapeDtypeStruct((Q, OUT), jnp.float32),
        scratch_types=[
            pltpu.VMEM((qpw * k,), jnp.int32),
            pltpu.VMEM((qpc * kc, OUTP), jnp.float32),
            pltpu.VMEM((qpc * kc, OUTP), jnp.float32),
            pltpu.VMEM((qpw, OUT), jnp.float32),
            pltpu.VMEM((qpw, OUT), jnp.float32),
            pltpu.SemaphoreType.DMA,
            pltpu.SemaphoreType.DMA,
        ],
    )
    def kern(table_hbm, idx_hbm, qx_hbm, out_hbm, idx_v, rows0_v, rows1_v,
             qx_v, out_v, sem0, sem1):
        wid = lax.axis_index("s") * NC + lax.axis_index("c")
        qbase0 = wid * qpw
        pltpu.sync_copy(idx_hbm.at[pl.ds(qbase0 * k, qpw * k)], idx_v)
        pltpu.sync_copy(qx_hbm.at[pl.ds(qbase0, qpw)], qx_v)
        bufs = [(rows0_v, sem0), (rows1_v, sem1)]

        def start(c):
            rv, sm = bufs[c % 2]
            off = (c // SUB) * qpc * k + (c % SUB) * kc
            return pltpu.async_copy(
                table_hbm.at[idx_v.at[pl.ds(off, qpc * kc)]], rv, sm)

        handle = start(0)
        for c in range(nchunks):
            nxt = start(c + 1) if c + 1 < nchunks else None
            handle.wait()
            rows_v = bufs[c % 2][0]
            qg, s = c // SUB, c % SUB

            def per_query(qi, c2, qg=qg, s=s, rows_v=rows_v):
                base = qi * kc
                qrow = qg * qpc + qi

                def per_co(co, c3):
                    sl = pl.ds(co * 16, 16)
                    vals = [rows_v[base + j, sl] for j in range(kc)]
                    while len(vals) > 1:
                        nxt2 = [jnp.maximum(vals[t], vals[t + 1])
                                for t in range(0, len(vals) - 1, 2)]
                        if len(vals) % 2:
                            nxt2.append(vals[-1])
                        vals = nxt2
                    part = vals[0]
                    if SUB == 2 and s == 0:
                        out_v[qrow, sl] = part
                    else:
                        if SUB == 2:
                            part = jnp.maximum(part, out_v[qrow, sl])
                        out_v[qrow, sl] = jnp.maximum(
                            part - qx_v[qrow, sl], 0.0)
                    return c3

                return lax.fori_loop(0, nco, per_co, c2)

            lax.fori_loop(0, qpc, per_query, 0)
            handle = nxt
        pltpu.sync_copy(out_v, out_hbm.at[pl.ds(qbase0, qpw)])

    return kern(table, idxf, qx)


def _cls_body(f_ref, W0_ref, b0_ref, W1_ref, b1_ref, W2_ref, b2_ref, o_ref):
    pooled = jnp.max(f_ref[...], axis=1)
    h = jnp.maximum(jnp.dot(pooled, W0_ref[...],
                            preferred_element_type=jnp.float32,
                            precision=lax.Precision.HIGHEST)
                    + b0_ref[...], 0.0)
    h = jnp.maximum(jnp.dot(h, W1_ref[...],
                            preferred_element_type=jnp.float32,
                            precision=lax.Precision.HIGHEST)
                    + b1_ref[...], 0.0)
    o_ref[...] = (jnp.dot(h, W2_ref[...], preferred_element_type=jnp.float32,
                          precision=lax.Precision.HIGHEST)
                  + b2_ref[...])


def kernel(x, W0, b0, W1, b1, W2, b2, W3, b3, Wc0, bc0, Wc1, bc1, Wc2, bc2):
    B = x.shape[0]
    xyz = jnp.transpose(x[:, :3, :], (0, 2, 1))
    normal = jnp.transpose(x[:, 3:, :], (0, 2, 1))
    feats = jnp.concatenate([xyz, normal], axis=-1)

    chunks = [1024, 512, 256, 128]
    params = [(W0, b0), (W1, b1), (W2, b2), (W3, b3)]
    for i, ((n_s, k), (W, b)) in enumerate(zip(_LEVELS, params)):
        N = feats.shape[1]
        OUT = W.shape[1]
        xyz, idx, pre, qx = _select(xyz, feats, W, b, n_s, k, chunks[i])
        out = _gather_max_sc(pre.reshape(B * N, -1), idx.reshape(-1),
                             qx.reshape(B * n_s, OUT), k)
        feats = out.reshape(B, n_s, OUT)

    logits = pl.pallas_call(
        _cls_body,
        out_shape=jax.ShapeDtypeStruct((B, Wc2.shape[1]), jnp.float32),
    )(feats, Wc0, bc0.reshape(1, -1), Wc1, bc1.reshape(1, -1),
      Wc2, bc2.reshape(1, -1))
    new_points = jnp.transpose(feats, (0, 2, 1))
    return logits, new_points
